# Initial kernel scaffold; baseline (speedup 1.0000x reference)
import jax
import jax.numpy as jnp
from jax.experimental import pallas as pl


def kernel(x, verts, edges, Wb, bb, W0_0, b0_0, W1_0, b1_0, W0_1, b0_1, W1_1, b1_1, W0_2, b0_2, W1_2, b1_2, Wo, bo):
    raise NotImplementedError("write your pallas kernel here")


# speedup vs baseline: 6.8908x; 6.8908x over previous
import functools

import jax
import jax.numpy as jnp
from jax import lax
from jax.experimental import pallas as pl
from jax.experimental.pallas import tpu as pltpu
from jax.experimental.pallas import tpu_sc as plsc

NC = 2
NS = 16
NW = NC * NS

V = 10000
E = 320000
VP = 10240
VPW = VP // NW
ACH = 64
ANCH = VPW // ACH
ROWS_PER_SUB = VP // NS

ECH = 128
EPW_CH = (2 * E + NW * ECH - 1) // (NW * ECH)
EPW = EPW_CH * ECH
EPAD = NW * EPW - 2 * E

FH = 64
FW = 64
CIMG = 256
HIDDEN = 128
SUPPORT = 64

_mesh = plsc.VectorSubcoreMesh(core_axis_name="c", subcore_axis_name="s")


def _align_body(table, vx, vy, out, vxv, vyv, idxv, wtsv, cb0, cb1, cb2, cb3,
                ob, s0, s1, s2, s3):
    c = lax.axis_index("c")
    s = lax.axis_index("s")
    wid = s * NC + c
    pltpu.sync_copy(vx.at[wid], vxv)
    pltpu.sync_copy(vy.at[wid], vyv)

    wmax = float(FW - 1)

    def comp(i, carry):
        xv = vxv[pl.ds(i * 16, 16)]
        yv = vyv[pl.ds(i * 16, 16)]
        px = jnp.minimum(jnp.maximum((xv + 1.0) * (0.5 * wmax), 0.0), wmax)
        py = jnp.minimum(jnp.maximum((yv + 1.0) * (0.5 * wmax), 0.0), wmax)
        x0 = px.astype(jnp.int32)
        y0 = py.astype(jnp.int32)
        wx = px - x0.astype(jnp.float32)
        wy = py - y0.astype(jnp.float32)
        x1 = jnp.minimum(x0 + 1, FW - 1)
        y1 = jnp.minimum(y0 + 1, FH - 1)
        r0 = y0 * FW
        r1 = y1 * FW
        sl = pl.ds(i * 16, 16)
        idxv[0, sl] = r0 + x0
        idxv[1, sl] = r0 + x1
        idxv[2, sl] = r1 + x0
        idxv[3, sl] = r1 + x1
        wtsv[0, sl] = (1.0 - wy) * (1.0 - wx)
        wtsv[1, sl] = (1.0 - wy) * wx
        wtsv[2, sl] = wy * (1.0 - wx)
        wtsv[3, sl] = wy * wx
        return carry

    lax.fori_loop(0, VPW // 16, comp, 0)

    cbs = (cb0, cb1, cb2, cb3)
    sems = (s0, s1, s2, s3)

    def chunk(k, carry):
        ds = [pltpu.async_copy(table.at[idxv.at[cc].at[pl.ds(k * ACH, ACH)]],
                               cbs[cc], sems[cc]) for cc in range(4)]
        for d in ds:
            d.wait()

        def comb(t, carry2):
            base = t * 16
            w0v = wtsv[0, pl.ds(k * ACH + base, 16)]
            w1v = wtsv[1, pl.ds(k * ACH + base, 16)]
            w2v = wtsv[2, pl.ds(k * ACH + base, 16)]
            w3v = wtsv[3, pl.ds(k * ACH + base, 16)]
            for lane in range(16):
                v = base + lane
                w0 = w0v[lane]
                w1 = w1v[lane]
                w2 = w2v[lane]
                w3 = w3v[lane]
                for c2 in range(CIMG // 16):
                    sl = pl.ds(c2 * 16, 16)
                    ob[v, sl] = (cb0[v, sl] * w0 + cb1[v, sl] * w1
                                 + cb2[v, sl] * w2 + cb3[v, sl] * w3)
            return carry2

        lax.fori_loop(0, ACH // 16, comb, 0)
        pltpu.sync_copy(ob, out.at[pl.ds(wid * VPW + k * ACH, ACH)])
        return carry

    lax.fori_loop(0, ANCH, chunk, 0)


_align_call = pl.kernel(
    _align_body,
    out_type=jax.ShapeDtypeStruct((VP, CIMG), jnp.float32),
    mesh=_mesh,
    scratch_types=[
        pltpu.VMEM((VPW,), jnp.float32),
        pltpu.VMEM((VPW,), jnp.float32),
        pltpu.VMEM((4, VPW), jnp.int32),
        pltpu.VMEM((4, VPW), jnp.float32),
        pltpu.VMEM((ACH, CIMG), jnp.float32),
        pltpu.VMEM((ACH, CIMG), jnp.float32),
        pltpu.VMEM((ACH, CIMG), jnp.float32),
        pltpu.VMEM((ACH, CIMG), jnp.float32),
        pltpu.VMEM((ACH, CIMG), jnp.float32),
        pltpu.SemaphoreType.DMA,
        pltpu.SemaphoreType.DMA,
        pltpu.SemaphoreType.DMA,
        pltpu.SemaphoreType.DMA,
    ],
)


def _scatter_body(w1t, srci, dsti, zrows, out, srcv, dstv, rows, acc, sem):
    c = lax.axis_index("c")
    s = lax.axis_index("s")
    wid = s * NC + c
    pltpu.sync_copy(zrows.at[pl.ds(s * ROWS_PER_SUB, ROWS_PER_SUB)],
                    acc.at[pl.ds(s * ROWS_PER_SUB, ROWS_PER_SUB)])
    pltpu.sync_copy(srci.at[wid], srcv)
    pltpu.sync_copy(dsti.at[wid], dstv)
    plsc.subcore_barrier()

    def body(j, carry):
        pltpu.async_copy(w1t.at[srcv.at[j]], rows, sem).wait()
        pltpu.sync_copy(rows, acc.at[dstv.at[j]], add=True)
        return carry

    lax.fori_loop(0, EPW_CH, body, 0)
    plsc.subcore_barrier()
    pltpu.sync_copy(acc.at[pl.ds(s * ROWS_PER_SUB, ROWS_PER_SUB)],
                    out.at[c].at[pl.ds(s * ROWS_PER_SUB, ROWS_PER_SUB)])


_scatter_call = pl.kernel(
    _scatter_body,
    out_type=jax.ShapeDtypeStruct((NC, VP, SUPPORT), jnp.float32),
    mesh=_mesh,
    compiler_params=pltpu.CompilerParams(use_tc_tiling_on_sc=False),
    scratch_types=[
        pltpu.VMEM((EPW_CH, ECH), jnp.int32),
        pltpu.VMEM((EPW_CH, ECH), jnp.int32),
        pltpu.VMEM((ECH, SUPPORT), jnp.float32),
        pltpu.VMEM_SHARED((VP, SUPPORT), jnp.float32),
        pltpu.SemaphoreType.DMA,
    ],
)


_BR = 1024
_P = jax.lax.Precision.HIGHEST


def _dot(a, b):
    return jnp.dot(a, b, precision=_P, preferred_element_type=jnp.float32)


def _tc1_body(f_ref, v8_ref, wbt_ref, bb_ref, w0at_ref, w0bt_ref, b0_ref,
              w1at_ref, w1bt_ref, b1_ref, w0_out, w1_out):
    imgf = jnp.maximum(_dot(f_ref[...], wbt_ref[...]) + bb_ref[...], 0.0)
    v8 = v8_ref[...]
    w0_out[...] = _dot(imgf, w0at_ref[...]) + _dot(v8, w0bt_ref[...]) + b0_ref[...]
    w1_out[...] = _dot(imgf, w1at_ref[...]) + _dot(v8, w1bt_ref[...]) + b1_ref[...]


def _mid_body(w0p_ref, p0_ref, p1_ref, v8_ref, w0at_ref, w0bt_ref, b0_ref,
              w1at_ref, w1bt_ref, b1_ref, w0_out, w1_out):
    ns = p0_ref[...] + p1_ref[...]
    w0p = w0p_ref[...]
    vf_a = jnp.maximum(w0p[:, :SUPPORT] + ns, 0.0)
    vf_b = jnp.maximum(w0p[:, SUPPORT:], 0.0)
    vf = jnp.concatenate([vf_a, vf_b], axis=1)
    v8 = v8_ref[...]
    w0_out[...] = _dot(vf, w0at_ref[...]) + _dot(v8, w0bt_ref[...]) + b0_ref[...]
    w1_out[...] = _dot(vf, w1at_ref[...]) + _dot(v8, w1bt_ref[...]) + b1_ref[...]


def _fin_body(w0p_ref, p0_ref, p1_ref, v8_ref, woat_ref, wobt_ref, bo_ref,
              vf_out, nv_out):
    ns = p0_ref[...] + p1_ref[...]
    w0p = w0p_ref[...]
    vf_a = jnp.maximum(w0p[:, :SUPPORT] + ns, 0.0)
    vf_b = jnp.maximum(w0p[:, SUPPORT:], 0.0)
    vf = jnp.concatenate([vf_a, vf_b], axis=1)
    vf_out[...] = vf
    v8 = v8_ref[...]
    d = jnp.tanh(_dot(vf, woat_ref[...]) + _dot(v8, wobt_ref[...]) + bo_ref[...])
    nv_out[...] = v8 + d


def _row_spec(cols):
    return pl.BlockSpec((_BR, cols), lambda i: (i, 0))


def _full_spec(shape):
    return pl.BlockSpec(shape, lambda i: tuple(0 for _ in shape))


def _tc1(f, v8, wbt, bb, w0at, w0bt, b0, w1at, w1bt, b1):
    return pl.pallas_call(
        _tc1_body,
        grid=(VP // _BR,),
        in_specs=[
            _row_spec(CIMG), _row_spec(8),
            _full_spec(wbt.shape), _full_spec(bb.shape),
            _full_spec(w0at.shape), _full_spec(w0bt.shape), _full_spec(b0.shape),
            _full_spec(w1at.shape), _full_spec(w1bt.shape), _full_spec(b1.shape),
        ],
        out_specs=[_row_spec(HIDDEN), _row_spec(SUPPORT)],
        out_shape=[
            jax.ShapeDtypeStruct((VP, HIDDEN), jnp.float32),
            jax.ShapeDtypeStruct((VP, SUPPORT), jnp.float32),
        ],
    )(f, v8, wbt, bb, w0at, w0bt, b0, w1at, w1bt, b1)


def _mid(w0p, p0, p1, v8, w0at, w0bt, b0, w1at, w1bt, b1):
    return pl.pallas_call(
        _mid_body,
        grid=(VP // _BR,),
        in_specs=[
            _row_spec(HIDDEN), _row_spec(SUPPORT), _row_spec(SUPPORT), _row_spec(8),
            _full_spec(w0at.shape), _full_spec(w0bt.shape), _full_spec(b0.shape),
            _full_spec(w1at.shape), _full_spec(w1bt.shape), _full_spec(b1.shape),
        ],
        out_specs=[_row_spec(HIDDEN), _row_spec(SUPPORT)],
        out_shape=[
            jax.ShapeDtypeStruct((VP, HIDDEN), jnp.float32),
            jax.ShapeDtypeStruct((VP, SUPPORT), jnp.float32),
        ],
    )(w0p, p0, p1, v8, w0at, w0bt, b0, w1at, w1bt, b1)


def _fin(w0p, p0, p1, v8, woat, wobt, bo):
    return pl.pallas_call(
        _fin_body,
        grid=(VP // _BR,),
        in_specs=[
            _row_spec(HIDDEN), _row_spec(SUPPORT), _row_spec(SUPPORT), _row_spec(8),
            _full_spec(woat.shape), _full_spec(wobt.shape), _full_spec(bo.shape),
        ],
        out_specs=[_row_spec(HIDDEN), _row_spec(8)],
        out_shape=[
            jax.ShapeDtypeStruct((VP, HIDDEN), jnp.float32),
            jax.ShapeDtypeStruct((VP, 8), jnp.float32),
        ],
    )(w0p, p0, p1, v8, woat, wobt, bo)


def kernel(x, verts, edges, Wb, bb, W0_0, b0_0, W1_0, b1_0, W0_1, b0_1,
           W1_1, b1_1, W0_2, b0_2, W1_2, b1_2, Wo, bo):
    f32 = jnp.float32

    table = x[0].transpose(1, 2, 0).reshape(FH * FW, CIMG)

    verts_p = jnp.zeros((VP, 8), f32).at[:V, :3].set(verts)
    vx = verts_p[:, 0].reshape(NW, VPW)
    vy = verts_p[:, 1].reshape(NW, VPW)

    pad_i = jnp.arange(EPAD, dtype=jnp.int32)
    src_all = jnp.concatenate([edges[:, 1], edges[:, 0], pad_i % VP])
    dst_all = jnp.concatenate([edges[:, 0], edges[:, 1], V + pad_i % (VP - V)])
    src_all = src_all.reshape(NW, EPW_CH, ECH)
    dst_all = dst_all.reshape(NW, EPW_CH, ECH)

    zrows = jnp.zeros((VP, SUPPORT), f32)

    def split_w(Wm, nout):
        wa = Wm[:, :HIDDEN].T
        wb = jnp.zeros((8, nout), f32).at[:3, :].set(Wm[:, HIDDEN:HIDDEN + 3].T)
        return wa, wb

    wbt = Wb.T
    bb2 = bb.reshape(1, HIDDEN)
    w0at_0, w0bt_0 = split_w(W0_0, HIDDEN)
    w1at_0, w1bt_0 = split_w(W1_0, SUPPORT)
    w0at_1, w0bt_1 = split_w(W0_1, HIDDEN)
    w1at_1, w1bt_1 = split_w(W1_1, SUPPORT)
    w0at_2, w0bt_2 = split_w(W0_2, HIDDEN)
    w1at_2, w1bt_2 = split_w(W1_2, SUPPORT)
    b0s = [b0_0.reshape(1, HIDDEN), b0_1.reshape(1, HIDDEN), b0_2.reshape(1, HIDDEN)]
    b1s = [b1_0.reshape(1, SUPPORT), b1_1.reshape(1, SUPPORT), b1_2.reshape(1, SUPPORT)]
    woat = jnp.zeros((HIDDEN, 8), f32).at[:, :3].set(Wo[:, :HIDDEN].T)
    wobt = jnp.zeros((8, 8), f32).at[:3, :3].set(Wo[:, HIDDEN:HIDDEN + 3].T)
    bo8 = jnp.zeros((1, 8), f32).at[0, :3].set(bo)

    feats = _align_call(table, vx, vy)

    w0, w1 = _tc1(feats, verts_p, wbt, bb2, w0at_0, w0bt_0, b0s[0],
                  w1at_0, w1bt_0, b1s[0])

    parts = _scatter_call(w1, src_all, dst_all, zrows)
    w0, w1 = _mid(w0, parts[0], parts[1], verts_p, w0at_1, w0bt_1, b0s[1],
                  w1at_1, w1bt_1, b1s[1])

    parts = _scatter_call(w1, src_all, dst_all, zrows)
    w0, w1 = _mid(w0, parts[0], parts[1], verts_p, w0at_2, w0bt_2, b0s[2],
                  w1at_2, w1bt_2, b1s[2])

    parts = _scatter_call(w1, src_all, dst_all, zrows)
    vf, nv8 = _fin(w0, parts[0], parts[1], verts_p, woat, wobt, bo8)

    return nv8[:V, :3], vf[:V]
